# baseline (speedup 1.0000x reference)
import functools

import jax
import jax.numpy as jnp
from jax.experimental import pallas as pl
from jax.experimental.pallas import tpu as pltpu

_NEG = float(jnp.finfo(jnp.float32).min)
_INDEX_TOPK = 2048


def _bitonic_stage(sk, si, col, j, k2):
    lower = (col & j) == 0
    pk = jnp.where(lower, jnp.roll(sk, -j, axis=1), jnp.roll(sk, j, axis=1))
    pi = jnp.where(lower, jnp.roll(si, -j, axis=1), jnp.roll(si, j, axis=1))
    pw = (pk > sk) | ((pk == sk) & (pi < si))
    take = pw ^ (~lower) ^ ((col & k2) != 0)
    return jnp.where(take, pk, sk), jnp.where(take, pi, si)


def _body(scal_ref, q_ref, k_ref, w_ref, idx_out_ref, val_out_ref,
          *, R, T, H, D, S, ratio, log2t):
    b = pl.program_id(0)
    seqlen = scal_ref[0]
    offset = scal_ref[1]

    q = q_ref[0].reshape(R * H, D)
    km = k_ref[0]
    s = jax.lax.dot_general(q, km, (((1,), (1,)), ((), ())),
                            preferred_element_type=jnp.float32)
    s = jnp.maximum(s, 0.0).reshape(R, H, T) * w_ref[0][:, :, None]
    s = s.sum(axis=1)

    row = b * R + jax.lax.broadcasted_iota(jnp.int32, (R, T), 0)
    col = jax.lax.broadcasted_iota(jnp.int32, (R, T), 1)
    thresh = (row + (seqlen - S) + 1) // ratio
    s = jnp.where(col >= thresh, _NEG, s)

    val_out_ref[0] = s
    idx_out_ref[0] = col

    maxth = ((b + 1) * R + (seqlen - S)) // ratio
    for p in range(1, log2t + 1):
        k2 = 1 << p

        @pl.when((k2 // 2) < maxth)
        def _phase(p=p, k2=k2):
            sk = val_out_ref[0]
            si = idx_out_ref[0]
            for q2 in range(p - 1, -1, -1):
                sk, si = _bitonic_stage(sk, si, col, 1 << q2, k2)
            val_out_ref[0] = sk
            idx_out_ref[0] = si

    si = idx_out_ref[0]
    idx_out_ref[0] = jnp.where(si >= thresh, -1, si + offset)


def _run(q_indexer, k_indexer, weights, seqlen, offset, interpret=False):
    B, S, H, D = q_indexer.shape
    T = k_indexer.shape[1]
    ratio = S // T
    k_out = min(_INDEX_TOPK, S // ratio)
    assert k_out == T, "kernel assumes full-width top_k (k == t)"
    log2t = T.bit_length() - 1
    assert (1 << log2t) == T

    R = min(64, S)
    NB = S // R

    scal = jnp.stack([jnp.asarray(seqlen, jnp.int32),
                      jnp.asarray(offset, jnp.int32)])

    grid_spec = pltpu.PrefetchScalarGridSpec(
        num_scalar_prefetch=1,
        grid=(NB,),
        in_specs=[
            pl.BlockSpec((1, R, H, D), lambda b, s_ref: (0, b, 0, 0)),
            pl.BlockSpec((1, T, D), lambda b, s_ref: (0, 0, 0)),
            pl.BlockSpec((1, R, H), lambda b, s_ref: (0, b, 0)),
        ],
        out_specs=[
            pl.BlockSpec((1, R, T), lambda b, s_ref: (0, b, 0)),
            pl.BlockSpec((1, R, T), lambda b, s_ref: (0, b, 0)),
        ],
    )

    body = functools.partial(_body, R=R, T=T, H=H, D=D, S=S,
                             ratio=ratio, log2t=log2t)
    idx_out, val_out = pl.pallas_call(
        body,
        grid_spec=grid_spec,
        out_shape=[
            jax.ShapeDtypeStruct((B, S, T), jnp.int32),
            jax.ShapeDtypeStruct((B, S, T), jnp.float32),
        ],
        interpret=interpret,
    )(scal, q_indexer, k_indexer, weights)
    return idx_out, val_out


def kernel(q_indexer, k_indexer, weights, seqlen, offset):
    return _run(q_indexer, k_indexer, weights, seqlen, offset)
